# Initial kernel scaffold; baseline (speedup 1.0000x reference)
import jax
import jax.numpy as jnp
from jax.experimental import pallas as pl


def kernel(adjacent_matrix, inp_seq, inp_seq_len, embedding, W_msg, b_msg, Wz, Uz, bz, Wr, Ur, br, Wh, Uh, bh, W_out, b_out):
    raise NotImplementedError("write your pallas kernel here")


# speedup vs baseline: 1.5525x; 1.5525x over previous
import functools

import jax
import jax.numpy as jnp
from jax import lax
from jax.experimental import pallas as pl
from jax.experimental.pallas import tpu as pltpu
from jax.experimental.pallas import tpu_sc as plsc

_B, _L, _H = 16, 512, 128
_STEPS = 3
_NC, _NS = 2, 16
_NW = _NC * _NS
_ROWS_PER_W = _B * _L // _NW
_CHUNK = 128
_NCH = _ROWS_PER_W // _CHUNK


def _sc_gather(table, idx2d):
    mesh = plsc.VectorSubcoreMesh(core_axis_name="c", subcore_axis_name="s")

    @functools.partial(
        pl.kernel,
        mesh=mesh,
        out_type=jax.ShapeDtypeStruct((_B * _L, _H), jnp.float32),
        scratch_types=[
            pltpu.VMEM((_NCH, _CHUNK), jnp.int32),
            pltpu.VMEM((_ROWS_PER_W, _H), jnp.float32),
            pltpu.SemaphoreType.DMA,
        ],
    )
    def gather_k(table_hbm, idx_hbm, out_hbm, idx_v, rows_v, sem):
        wid = lax.axis_index("s") * _NC + lax.axis_index("c")
        pltpu.sync_copy(idx_hbm.at[pl.ds(wid * _NCH, _NCH)], idx_v)
        copies = [
            pltpu.async_copy(
                table_hbm.at[idx_v.at[j]],
                rows_v.at[pl.ds(j * _CHUNK, _CHUNK)],
                sem,
            )
            for j in range(_NCH)
        ]
        for cp in copies:
            cp.wait()
        pltpu.sync_copy(rows_v, out_hbm.at[pl.ds(wid * _ROWS_PER_W, _ROWS_PER_W)])

    return gather_k(table, idx2d)


def _ggnn_body(len_ref, bout_ref, a_ref, h_ref, wm_ref, wz_ref, uz_ref,
               wr_ref, ur_ref, wh_ref, uh_ref, bias_ref, wout_ref, out_ref):
    b = pl.program_id(0)
    n = len_ref[b, 0]
    mask = (lax.broadcasted_iota(jnp.int32, (_L, 1), 0) < n).astype(jnp.float32)
    h = h_ref[0, :, :] * mask
    a = a_ref[0, :, :]
    deg = jnp.clip(jnp.sum(a, axis=-1, keepdims=True), 1e-6, None)
    an = a / deg

    wm = wm_ref[...]
    wz, uz = wz_ref[...], uz_ref[...]
    wr, ur = wr_ref[...], ur_ref[...]
    wh, uh = wh_ref[...], uh_ref[...]
    b_msg = bias_ref[0:1, :]
    bz = bias_ref[1:2, :]
    br = bias_ref[2:3, :]
    bh = bias_ref[3:4, :]

    def mm(x, w):
        return jnp.dot(x, w, preferred_element_type=jnp.float32)

    for _ in range(_STEPS):
        m = mm(an, mm(h, wm)) + b_msg
        z = jax.nn.sigmoid(mm(m, wz) + mm(h, uz) + bz)
        r = jax.nn.sigmoid(mm(m, wr) + mm(h, ur) + br)
        hh = jnp.tanh(mm(m, wh) + mm(r * h, uh) + bh)
        h = ((1.0 - z) * h + z * hh) * mask

    logits = jnp.sum(h * wout_ref[0:1, :], axis=1) + bout_ref[0]
    out_ref[0, 0, :] = logits


def _tc_ggnn(adj, h0, seq_len, W_msg, Wz, Uz, Wr, Ur, Wh, Uh, biases,
             w_out_row, b_out):
    hmat = pl.BlockSpec((_H, _H), lambda b: (0, 0))
    return pl.pallas_call(
        _ggnn_body,
        grid=(_B,),
        in_specs=[
            pl.BlockSpec(memory_space=pltpu.SMEM),
            pl.BlockSpec(memory_space=pltpu.SMEM),
            pl.BlockSpec((1, _L, _L), lambda b: (b, 0, 0)),
            pl.BlockSpec((1, _L, _H), lambda b: (b, 0, 0)),
            hmat, hmat, hmat, hmat, hmat, hmat, hmat,
            pl.BlockSpec((4, _H), lambda b: (0, 0)),
            pl.BlockSpec((1, _H), lambda b: (0, 0)),
        ],
        out_specs=pl.BlockSpec((1, 1, _L), lambda b: (b, 0, 0)),
        out_shape=jax.ShapeDtypeStruct((_B, 1, _L), jnp.float32),
        compiler_params=pltpu.CompilerParams(
            dimension_semantics=("arbitrary",),
        ),
    )(seq_len, b_out, adj, h0, W_msg, Wz, Uz, Wr, Ur, Wh, Uh, biases,
      w_out_row)


def kernel(adjacent_matrix, inp_seq, inp_seq_len, embedding, W_msg, b_msg,
           Wz, Uz, bz, Wr, Ur, br, Wh, Uh, bh, W_out, b_out):
    idx2d = inp_seq.astype(jnp.int32).reshape(_NW * _NCH, _CHUNK)
    h_flat = _sc_gather(embedding, idx2d)
    h0 = h_flat.reshape(_B, _L, _H)
    biases = jnp.stack([b_msg, bz, br, bh])
    seq_len = inp_seq_len.astype(jnp.int32).reshape(_B, 1)
    w_out_row = W_out.reshape(1, _H)
    out3 = _tc_ggnn(adjacent_matrix, h0, seq_len, W_msg, Wz, Uz, Wr, Ur,
                    Wh, Uh, biases, w_out_row, b_out)
    return out3.reshape(_B, _L)
